# baseline (speedup 1.0000x reference)
import functools

import jax
import jax.numpy as jnp
from jax import lax
from jax.experimental import pallas as pl
from jax.experimental.pallas import tpu as pltpu

E = 8
KTH = 0.8
CAPF = 1.5
BAL = 0.005
REG = 1e-06
DOUT = 256

_HI = jax.lax.Precision.HIGHEST
_DEF = jax.lax.Precision.DEFAULT


def _dot(a, b, dims, precision=_DEF):
    return lax.dot_general(a, b, (dims, ((), ())), precision=precision,
                           preferred_element_type=jnp.float32)


def _exact_dot(a, b, dims):
    a_hi = a.astype(jnp.bfloat16).astype(jnp.float32)
    return _dot(a_hi, b, dims) + _dot(a - a_hi, b, dims)


def _gelu(t):
    return 0.5 * t * (1.0 + lax.erf(t * jnp.float32(0.7071067811865476)))


def _conv_pool_kernel(L, G, x_ref, w_ref, b_ref, f_ref, xm_ref):
    x = x_ref[...]
    W = w_ref[...]
    nco = W.shape[1]
    y0 = _dot(W[0], x, ((1,), (1,)))
    y1 = _dot(W[1], x, ((1,), (1,)))
    y2 = _dot(W[2], x, ((1,), (1,)))
    li = lax.broadcasted_iota(jnp.int32, y0.shape, 2)
    h = y1
    h = h + jnp.where(li == 0, 0.0, jnp.roll(y0, 1, axis=2))
    h = h + jnp.where(li == L - 1, 0.0, jnp.roll(y2, -1, axis=2))
    h = h + b_ref[...][:, :, None]
    h = _gelu(h)
    pi = lax.broadcasted_iota(jnp.int32, (L, 16), 0)
    pg = lax.broadcasted_iota(jnp.int32, (L, 16), 1)
    P = jnp.where((pi // 16) == pg, 1.0 / 16.0, 0.0).astype(jnp.float32)
    f_ref[...] = _exact_dot(h, P, ((2,), (0,)))
    xm_ref[...] = jnp.mean(x, axis=2)


def _routing(B, cap, f, bng, bnb, w1, b1, w2, b2):
    fb = f / jnp.sqrt(jnp.float32(1.0 + 1e-05)) * bng + bnb
    a1 = _gelu(_dot(fb, w1, ((1,), (0,))) + b1)
    logits = _dot(a1, w2, ((1,), (0,))) + b2
    m = jnp.max(logits, axis=1, keepdims=True)
    ex = jnp.exp(logits - m)
    w = ex / jnp.sum(ex, axis=1, keepdims=True)

    eidx = lax.broadcasted_iota(jnp.int32, (B, E), 1)
    r = jnp.zeros((B, E), jnp.float32)
    for j in range(E):
        colj = w[:, j:j + 1]
        r = r + ((colj > w) | ((colj == w) & (j < eidx))).astype(jnp.float32)
    kcnt = jnp.zeros((B, 1), jnp.float32)
    for mm in range(1, E + 1):
        Sm = jnp.sum(jnp.where(r < mm, w, 0.0), axis=1, keepdims=True)
        kcnt = kcnt + (Sm <= KTH).astype(jnp.float32)
    k = jnp.clip(1.0 + kcnt, 1.0, float(E))
    maxk = jnp.max(k, axis=(0, 1), keepdims=True)
    valid = r < maxk
    validf = valid.astype(jnp.float32)

    i8a = lax.broadcasted_iota(jnp.int32, (E, E), 0)
    i8b = lax.broadcasted_iota(jnp.int32, (E, E), 1)
    I8 = (i8a == i8b).astype(jnp.float32)
    wT = _dot(I8, w, ((1,), (1,)), precision=_HI)
    vT = _dot(I8, validf, ((1,), (1,)), precision=_HI)

    keep_cols = []
    for e in range(E):
        roww = wT[e:e + 1, :]
        rowv = vT[e:e + 1, :]
        cmp = (roww > w[:, e:e + 1]) & (rowv > 0.5)
        rk = jnp.sum(cmp.astype(jnp.float32), axis=1, keepdims=True)
        keep_cols.append(
            (valid[:, e:e + 1] & (rk < cap)).astype(jnp.float32))
    keepf = jnp.concatenate(keep_cols, axis=1)
    cnt = jnp.sum(keepf, axis=0, keepdims=True)

    z = w * keepf
    zm = jnp.max(z, axis=1, keepdims=True)
    ez = jnp.where(valid, jnp.exp(z - zm), 0.0)
    nw = ez / jnp.sum(ez, axis=1, keepdims=True)
    c = keepf * nw

    us = jnp.mean(w, axis=0, keepdims=True)
    mu = jnp.mean(us, axis=1, keepdims=True)
    var = jnp.sum((us - mu) ** 2, axis=1, keepdims=True) / (E - 1)
    return c, cnt, us, var


def _route_expert_kernel(B, cap, f_ref, bng_ref, bnb_ref, rw1_ref,
                         rb1_ref, rw2_ref, rb2_ref, xm_ref, w1_ref,
                         b1_ref, w2_ref, b2_ref, out_ref, reg_ref,
                         c_ref, cnt_ref, us_ref, var_ref):
    e = pl.program_id(0)

    @pl.when(e == 0)
    def _():
        c, cnt, us, var = _routing(B, cap, f_ref[...], bng_ref[...],
                                   bnb_ref[...], rw1_ref[...], rb1_ref[...],
                                   rw2_ref[...], rb2_ref[...])
        c_ref[...] = c
        cnt_ref[...] = cnt
        us_ref[...] = us
        var_ref[...] = var

    W1 = w1_ref[0]
    b1 = b1_ref[0]
    W2 = w2_ref[0]
    b2 = b2_ref[0]
    H = _gelu(_dot(xm_ref[...], W1, ((1,), (0,))) + b1)
    O = _dot(H, W2, ((1,), (0,))) + b2
    eidx = lax.broadcasted_iota(jnp.int32, (B, E), 1)
    ce = jnp.sum(jnp.where(eidx == e, c_ref[...], 0.0), axis=1,
                 keepdims=True)
    contrib = ce * O
    part = (jnp.sum(W1 * W1, keepdims=True).reshape(1, 1)
            + jnp.sum(b1 * b1, keepdims=True).reshape(1, 1)
            + jnp.sum(W2 * W2, keepdims=True).reshape(1, 1)
            + jnp.sum(b2 * b2, keepdims=True).reshape(1, 1))

    @pl.when(e == 0)
    def _():
        out_ref[...] = contrib
        reg_ref[...] = part

    @pl.when(e != 0)
    def _():
        out_ref[...] = out_ref[...] + contrib
        reg_ref[...] = reg_ref[...] + part


def kernel(x, conv_w, conv_b, bn_g, bn_b, lin1_w, lin1_b, lin2_w, lin2_b,
           eW1, eb1, eW2, eb2):
    B, C, L = x.shape
    cap = max(1, int(B * CAPF / E))
    G = 64
    nco = conv_w.shape[0]
    npool = L // 16
    fdim = nco * 16

    conv_wT = conv_w.transpose(2, 0, 1)
    conv_b2 = conv_b.reshape(nco, 1)

    fT, xm = pl.pallas_call(
        functools.partial(_conv_pool_kernel, L, G),
        grid=(B // G,),
        in_specs=[
            pl.BlockSpec((G, C, L), lambda i: (i, 0, 0)),
            pl.BlockSpec((3, nco, C), lambda i: (0, 0, 0)),
            pl.BlockSpec((nco, 1), lambda i: (0, 0)),
        ],
        out_specs=[
            pl.BlockSpec((nco, G, 16), lambda i: (0, i, 0)),
            pl.BlockSpec((G, C), lambda i: (i, 0)),
        ],
        out_shape=[
            jax.ShapeDtypeStruct((nco, B, 16), jnp.float32),
            jax.ShapeDtypeStruct((B, C), jnp.float32),
        ],
    )(x, conv_wT, conv_b2)

    f = fT.transpose(1, 0, 2).reshape(B, fdim)

    hdim = eW1.shape[2]
    weighted, reg, c, cnt, us, var = pl.pallas_call(
        functools.partial(_route_expert_kernel, B, cap),
        grid=(E,),
        in_specs=[
            pl.BlockSpec((B, fdim), lambda e: (0, 0)),
            pl.BlockSpec((1, fdim), lambda e: (0, 0)),
            pl.BlockSpec((1, fdim), lambda e: (0, 0)),
            pl.BlockSpec((fdim, 128), lambda e: (0, 0)),
            pl.BlockSpec((1, 128), lambda e: (0, 0)),
            pl.BlockSpec((128, E), lambda e: (0, 0)),
            pl.BlockSpec((1, E), lambda e: (0, 0)),
            pl.BlockSpec((B, C), lambda e: (0, 0)),
            pl.BlockSpec((1, C, hdim), lambda e: (e, 0, 0)),
            pl.BlockSpec((1, 1, hdim), lambda e: (e, 0, 0)),
            pl.BlockSpec((1, hdim, DOUT), lambda e: (e, 0, 0)),
            pl.BlockSpec((1, 1, DOUT), lambda e: (e, 0, 0)),
        ],
        out_specs=[
            pl.BlockSpec((B, DOUT), lambda e: (0, 0)),
            pl.BlockSpec((1, 1), lambda e: (0, 0)),
            pl.BlockSpec((B, E), lambda e: (0, 0)),
            pl.BlockSpec((1, E), lambda e: (0, 0)),
            pl.BlockSpec((1, E), lambda e: (0, 0)),
            pl.BlockSpec((1, 1), lambda e: (0, 0)),
        ],
        out_shape=[
            jax.ShapeDtypeStruct((B, DOUT), jnp.float32),
            jax.ShapeDtypeStruct((1, 1), jnp.float32),
            jax.ShapeDtypeStruct((B, E), jnp.float32),
            jax.ShapeDtypeStruct((1, E), jnp.float32),
            jax.ShapeDtypeStruct((1, E), jnp.float32),
            jax.ShapeDtypeStruct((1, 1), jnp.float32),
        ],
    )(f, bn_g.reshape(1, -1), bn_b.reshape(1, -1), lin1_w.T,
      lin1_b.reshape(1, -1), lin2_w.T, lin2_b.reshape(1, -1),
      xm, eW1, eb1.reshape(E, 1, hdim), eW2, eb2.reshape(E, 1, DOUT))

    added = BAL * var[0, 0] + REG * reg[0, 0]
    return (weighted, added, us.reshape(E), cnt.reshape(E))


# speedup vs baseline: 1.0245x; 1.0245x over previous
import functools

import jax
import jax.numpy as jnp
from jax import lax
from jax.experimental import pallas as pl
from jax.experimental.pallas import tpu as pltpu

E = 8
KTH = 0.8
CAPF = 1.5
BAL = 0.005
REG = 1e-06
DOUT = 256

_HI = jax.lax.Precision.HIGHEST
_DEF = jax.lax.Precision.DEFAULT


def _dot(a, b, dims, precision=_DEF):
    return lax.dot_general(a, b, (dims, ((), ())), precision=precision,
                           preferred_element_type=jnp.float32)


def _exact_dot(a, b, dims):
    a_hi = a.astype(jnp.bfloat16).astype(jnp.float32)
    return _dot(a_hi, b, dims) + _dot(a - a_hi, b, dims)


def _gelu(t):
    return 0.5 * t * (1.0 + lax.erf(t * jnp.float32(0.7071067811865476)))


def _conv_pool_kernel(L, G, x_ref, w_ref, b_ref, f_ref, xm_ref):
    x = x_ref[...]
    W = w_ref[...]
    nco = W.shape[1]
    y0 = _dot(W[0], x, ((1,), (1,)))
    y1 = _dot(W[1], x, ((1,), (1,)))
    y2 = _dot(W[2], x, ((1,), (1,)))
    li = lax.broadcasted_iota(jnp.int32, y0.shape, 2)
    h = y1
    h = h + jnp.where(li == 0, 0.0, jnp.roll(y0, 1, axis=2))
    h = h + jnp.where(li == L - 1, 0.0, jnp.roll(y2, -1, axis=2))
    h = h + b_ref[...][:, :, None]
    h = _gelu(h)
    pi = lax.broadcasted_iota(jnp.int32, (L, 16), 0)
    pg = lax.broadcasted_iota(jnp.int32, (L, 16), 1)
    P = jnp.where((pi // 16) == pg, 1.0 / 16.0, 0.0).astype(jnp.float32)
    f_ref[...] = _exact_dot(h, P, ((2,), (0,)))
    xm_ref[...] = jnp.mean(x, axis=2)


def _routing(B, cap, f, bng, bnb, w1, b1, w2, b2):
    fb = f / jnp.sqrt(jnp.float32(1.0 + 1e-05)) * bng + bnb
    a1 = _gelu(_dot(fb, w1, ((1,), (0,))) + b1)
    logits = _dot(a1, w2, ((1,), (0,))) + b2
    m = jnp.max(logits, axis=1, keepdims=True)
    ex = jnp.exp(logits - m)
    w = ex / jnp.sum(ex, axis=1, keepdims=True)

    eidx = lax.broadcasted_iota(jnp.int32, (B, E), 1)
    r = jnp.zeros((B, E), jnp.float32)
    for j in range(E):
        colj = w[:, j:j + 1]
        r = r + ((colj > w) | ((colj == w) & (j < eidx))).astype(jnp.float32)
    kcnt = jnp.zeros((B, 1), jnp.float32)
    for mm in range(1, E + 1):
        Sm = jnp.sum(jnp.where(r < mm, w, 0.0), axis=1, keepdims=True)
        kcnt = kcnt + (Sm <= KTH).astype(jnp.float32)
    k = jnp.clip(1.0 + kcnt, 1.0, float(E))
    maxk = jnp.max(k, axis=(0, 1), keepdims=True)
    valid = r < maxk
    validf = valid.astype(jnp.float32)

    i8a = lax.broadcasted_iota(jnp.int32, (E, E), 0)
    i8b = lax.broadcasted_iota(jnp.int32, (E, E), 1)
    I8 = (i8a == i8b).astype(jnp.float32)
    wT = _dot(I8, w, ((1,), (1,)), precision=_HI)
    vT = _dot(I8, validf, ((1,), (1,)), precision=_HI)

    keep_cols = []
    for e in range(E):
        roww = wT[e:e + 1, :]
        rowv = vT[e:e + 1, :]
        cmp = (roww > w[:, e:e + 1]) & (rowv > 0.5)
        rk = jnp.sum(cmp.astype(jnp.float32), axis=1, keepdims=True)
        keep_cols.append(
            (valid[:, e:e + 1] & (rk < cap)).astype(jnp.float32))
    keepf = jnp.concatenate(keep_cols, axis=1)
    cnt = jnp.sum(keepf, axis=0, keepdims=True)

    z = w * keepf
    zm = jnp.max(z, axis=1, keepdims=True)
    ez = jnp.where(valid, jnp.exp(z - zm), 0.0)
    nw = ez / jnp.sum(ez, axis=1, keepdims=True)
    c = keepf * nw

    us = jnp.mean(w, axis=0, keepdims=True)
    mu = jnp.mean(us, axis=1, keepdims=True)
    var = jnp.sum((us - mu) ** 2, axis=1, keepdims=True) / (E - 1)
    return c, cnt, us, var


def _route_expert_kernel(B, cap, f_ref, bng_ref, bnb_ref, rw1_ref,
                         rb1_ref, rw2_ref, rb2_ref, xm_ref, w1_ref,
                         b1_ref, w2_ref, b2_ref, out_ref, reg_ref,
                         c_ref, cnt_ref, us_ref, var_ref):
    e = pl.program_id(0)

    @pl.when(e == 0)
    def _():
        c, cnt, us, var = _routing(B, cap, f_ref[...], bng_ref[...],
                                   bnb_ref[...], rw1_ref[...], rb1_ref[...],
                                   rw2_ref[...], rb2_ref[...])
        c_ref[...] = c
        cnt_ref[...] = cnt
        us_ref[...] = us
        var_ref[...] = var

    W1 = w1_ref[0]
    b1 = b1_ref[0]
    W2 = w2_ref[0]
    b2 = b2_ref[0]
    H = _gelu(_dot(xm_ref[...], W1, ((1,), (0,))) + b1)
    O = _dot(H, W2, ((1,), (0,))) + b2
    eidx = lax.broadcasted_iota(jnp.int32, (B, E), 1)
    ce = jnp.sum(jnp.where(eidx == e, c_ref[...], 0.0), axis=1,
                 keepdims=True)
    contrib = ce * O
    part = (jnp.sum(W1 * W1, keepdims=True).reshape(1, 1)
            + jnp.sum(b1 * b1, keepdims=True).reshape(1, 1)
            + jnp.sum(W2 * W2, keepdims=True).reshape(1, 1)
            + jnp.sum(b2 * b2, keepdims=True).reshape(1, 1))

    @pl.when(e == 0)
    def _():
        out_ref[...] = contrib
        reg_ref[...] = part

    @pl.when(e != 0)
    def _():
        out_ref[...] = out_ref[...] + contrib
        reg_ref[...] = reg_ref[...] + part


def kernel(x, conv_w, conv_b, bn_g, bn_b, lin1_w, lin1_b, lin2_w, lin2_b,
           eW1, eb1, eW2, eb2):
    B, C, L = x.shape
    cap = max(1, int(B * CAPF / E))
    G = 32
    nco = conv_w.shape[0]
    npool = L // 16
    fdim = nco * 16

    conv_wT = conv_w.transpose(2, 0, 1)
    conv_b2 = conv_b.reshape(nco, 1)

    fT, xm = pl.pallas_call(
        functools.partial(_conv_pool_kernel, L, G),
        grid=(B // G,),
        in_specs=[
            pl.BlockSpec((G, C, L), lambda i: (i, 0, 0)),
            pl.BlockSpec((3, nco, C), lambda i: (0, 0, 0)),
            pl.BlockSpec((nco, 1), lambda i: (0, 0)),
        ],
        out_specs=[
            pl.BlockSpec((nco, G, 16), lambda i: (0, i, 0)),
            pl.BlockSpec((G, C), lambda i: (i, 0)),
        ],
        out_shape=[
            jax.ShapeDtypeStruct((nco, B, 16), jnp.float32),
            jax.ShapeDtypeStruct((B, C), jnp.float32),
        ],
    )(x, conv_wT, conv_b2)

    f = fT.transpose(1, 0, 2).reshape(B, fdim)

    hdim = eW1.shape[2]
    weighted, reg, c, cnt, us, var = pl.pallas_call(
        functools.partial(_route_expert_kernel, B, cap),
        grid=(E,),
        in_specs=[
            pl.BlockSpec((B, fdim), lambda e: (0, 0)),
            pl.BlockSpec((1, fdim), lambda e: (0, 0)),
            pl.BlockSpec((1, fdim), lambda e: (0, 0)),
            pl.BlockSpec((fdim, 128), lambda e: (0, 0)),
            pl.BlockSpec((1, 128), lambda e: (0, 0)),
            pl.BlockSpec((128, E), lambda e: (0, 0)),
            pl.BlockSpec((1, E), lambda e: (0, 0)),
            pl.BlockSpec((B, C), lambda e: (0, 0)),
            pl.BlockSpec((1, C, hdim), lambda e: (e, 0, 0)),
            pl.BlockSpec((1, 1, hdim), lambda e: (e, 0, 0)),
            pl.BlockSpec((1, hdim, DOUT), lambda e: (e, 0, 0)),
            pl.BlockSpec((1, 1, DOUT), lambda e: (e, 0, 0)),
        ],
        out_specs=[
            pl.BlockSpec((B, DOUT), lambda e: (0, 0)),
            pl.BlockSpec((1, 1), lambda e: (0, 0)),
            pl.BlockSpec((B, E), lambda e: (0, 0)),
            pl.BlockSpec((1, E), lambda e: (0, 0)),
            pl.BlockSpec((1, E), lambda e: (0, 0)),
            pl.BlockSpec((1, 1), lambda e: (0, 0)),
        ],
        out_shape=[
            jax.ShapeDtypeStruct((B, DOUT), jnp.float32),
            jax.ShapeDtypeStruct((1, 1), jnp.float32),
            jax.ShapeDtypeStruct((B, E), jnp.float32),
            jax.ShapeDtypeStruct((1, E), jnp.float32),
            jax.ShapeDtypeStruct((1, E), jnp.float32),
            jax.ShapeDtypeStruct((1, 1), jnp.float32),
        ],
    )(f, bn_g.reshape(1, -1), bn_b.reshape(1, -1), lin1_w.T,
      lin1_b.reshape(1, -1), lin2_w.T, lin2_b.reshape(1, -1),
      xm, eW1, eb1.reshape(E, 1, hdim), eW2, eb2.reshape(E, 1, DOUT))

    added = BAL * var[0, 0] + REG * reg[0, 0]
    return (weighted, added, us.reshape(E), cnt.reshape(E))
